# Initial kernel scaffold; baseline (speedup 1.0000x reference)
import jax
import jax.numpy as jnp
from jax.experimental import pallas as pl


def kernel(label, cls_ctx, token_prefix, token_suffix):
    raise NotImplementedError("write your pallas kernel here")


# speedup vs baseline: 1.0208x; 1.0208x over previous
import functools

import jax
import jax.numpy as jnp
from jax import lax
from jax.experimental import pallas as pl
from jax.experimental.pallas import tpu as pltpu
from jax.experimental.pallas import tpu_sc as plsc

NUM_CLASS = 1000
N_CLS_CTX = 4
CTX_DIM = 512
PREFIX_LEN = 6
SUFFIX_LEN = 67
SEQ_LEN = PREFIX_LEN + N_CLS_CTX + SUFFIX_LEN
BATCH = 1024

NC = 2
NS = 16
NW = NC * NS
BPW = BATCH // NW


GCHUNK = 16


def _body(cls_hbm, idx_hbm, tmpl_hbm, out_hbm,
          idx_v, rows_v, prompt_v, gsem):
    wid = lax.axis_index("s") * NC + lax.axis_index("c")
    base = wid * BPW

    pltpu.sync_copy(idx_hbm.at[pl.ds(base, BPW)], idx_v)

    pltpu.sync_copy(tmpl_hbm, prompt_v)

    for h in range(BPW // GCHUNK):
        pltpu.async_copy(
            cls_hbm.at[idx_v.at[pl.ds(h * GCHUNK, GCHUNK)]], rows_v, gsem,
        ).wait()

        def step(j, carry):
            for r in range(N_CLS_CTX):
                for c in range(CTX_DIM // 16):
                    prompt_v[PREFIX_LEN + r, pl.ds(c * 16, 16)] = (
                        rows_v[j, r, pl.ds(c * 16, 16)])
            pltpu.sync_copy(prompt_v, out_hbm.at[base + h * GCHUNK + j])
            return carry

        lax.fori_loop(0, GCHUNK, step, 0)


@jax.jit
def _prompt_learner(label, cls_ctx, tmpl):
    mesh = plsc.VectorSubcoreMesh(core_axis_name="c", subcore_axis_name="s")
    return pl.kernel(
        _body,
        out_type=jax.ShapeDtypeStruct((BATCH, SEQ_LEN, CTX_DIM), jnp.float32),
        mesh=mesh,
        scratch_types=[
            pltpu.VMEM((BPW,), jnp.int32),
            pltpu.VMEM((GCHUNK, N_CLS_CTX, CTX_DIM), jnp.float32),
            pltpu.VMEM((SEQ_LEN, CTX_DIM), jnp.float32),
            pltpu.SemaphoreType.DMA,
        ],
    )(cls_ctx, label, tmpl)


def kernel(label, cls_ctx, token_prefix, token_suffix):
    label = label.astype(jnp.int32)
    tmpl = jnp.concatenate(
        [token_prefix.reshape(PREFIX_LEN, CTX_DIM),
         jnp.zeros((N_CLS_CTX, CTX_DIM), jnp.float32),
         token_suffix.reshape(SUFFIX_LEN, CTX_DIM)], axis=0)
    return _prompt_learner(label, cls_ctx, tmpl)
